# Initial kernel scaffold; baseline (speedup 1.0000x reference)
import jax
import jax.numpy as jnp
from jax.experimental import pallas as pl


def kernel(x, edge_index, u, batch, W_rel0, b0, W_root0, W_rel1, b1, W_root1, W_rel2, b2, W_root2):
    raise NotImplementedError("write your pallas kernel here")


# speedup vs baseline: 3.3457x; 3.3457x over previous
import functools

import jax
import jax.numpy as jnp
from jax import lax
from jax.experimental import pallas as pl
from jax.experimental.pallas import tpu as pltpu
from jax.experimental.pallas import tpu_sc as plsc

N = 10000
E = 320000
F = 128
H = 256
G = 16
HH = H // 2
NS = 16
EPS = E // NS
CH = 80
NCHUNK = EPS // CH
RPS = 624
REM = N - NS * RPS
REM0 = NS * RPS
BLK = 2000


def _sc_body(p2, gidx, dsts, r_lo, r_hi, out_lo, out_hi,
             agg, idx_v, dst_v, rows_v, sem):
    c = lax.axis_index("c")
    s = lax.axis_index("s")
    row0 = s * RPS

    @pl.when(c == 0)
    def _():
        pltpu.sync_copy(r_lo.at[pl.ds(row0, RPS)], agg.at[pl.ds(row0, RPS)])

        @pl.when(s == NS - 1)
        def _():
            pltpu.sync_copy(r_lo.at[pl.ds(REM0, REM)], agg.at[pl.ds(REM0, REM)])

    @pl.when(c == 1)
    def _():
        pltpu.sync_copy(r_hi.at[pl.ds(row0, RPS)], agg.at[pl.ds(row0, RPS)])

        @pl.when(s == NS - 1)
        def _():
            pltpu.sync_copy(r_hi.at[pl.ds(REM0, REM)], agg.at[pl.ds(REM0, REM)])

    plsc.subcore_barrier()

    def chunk(k, carry):
        pltpu.sync_copy(gidx.at[c, s, k], idx_v)
        pltpu.sync_copy(dsts.at[s, k], dst_v)
        pltpu.async_copy(p2.at[idx_v], rows_v, sem).wait()
        pltpu.sync_copy(rows_v, agg.at[dst_v], add=True)
        return carry

    lax.fori_loop(0, NCHUNK, chunk, 0)
    plsc.subcore_barrier()

    @pl.when(c == 0)
    def _():
        pltpu.sync_copy(agg.at[pl.ds(row0, RPS)], out_lo.at[pl.ds(row0, RPS)])

        @pl.when(s == NS - 1)
        def _():
            pltpu.sync_copy(agg.at[pl.ds(REM0, REM)], out_lo.at[pl.ds(REM0, REM)])

    @pl.when(c == 1)
    def _():
        pltpu.sync_copy(agg.at[pl.ds(row0, RPS)], out_hi.at[pl.ds(row0, RPS)])

        @pl.when(s == NS - 1)
        def _():
            pltpu.sync_copy(agg.at[pl.ds(REM0, REM)], out_hi.at[pl.ds(REM0, REM)])


_sc_segsum = functools.partial(
    pl.kernel,
    out_type=(
        jax.ShapeDtypeStruct((N, HH), jnp.float32),
        jax.ShapeDtypeStruct((N, HH), jnp.float32),
    ),
    mesh=plsc.VectorSubcoreMesh(core_axis_name="c", subcore_axis_name="s"),
    scratch_types=[
        pltpu.VMEM_SHARED((N, HH), jnp.float32),
        pltpu.VMEM((CH,), jnp.int32),
        pltpu.VMEM((CH,), jnp.int32),
        pltpu.VMEM((CH, HH), jnp.float32),
        pltpu.SemaphoreType.DMA,
    ],
)(_sc_body)


def _tc0_body(x_r, bat_r, u_r, wrx_r, wru_r, wox_r, wou_r, b_r,
              p_r, rlo_r, rhi_r):
    xb = x_r[...]
    oh = (bat_r[...] == lax.broadcasted_iota(jnp.int32, (1, G), 1)
          ).astype(jnp.float32)
    uw_rel = jnp.dot(u_r[...], wru_r[...], preferred_element_type=jnp.float32)
    uw_root = jnp.dot(u_r[...], wou_r[...], preferred_element_type=jnp.float32)
    p = (jnp.dot(xb, wrx_r[...], preferred_element_type=jnp.float32)
         + jnp.dot(oh, uw_rel, preferred_element_type=jnp.float32))
    r = (jnp.dot(xb, wox_r[...], preferred_element_type=jnp.float32)
         + jnp.dot(oh, uw_root, preferred_element_type=jnp.float32)
         + b_r[...])
    p_r[...] = p
    rlo_r[...] = r[:, :HH]
    rhi_r[...] = r[:, HH:]


def _tc0(x, bat2, u_pad, wrx, wru, wox, wou, b2):
    return pl.pallas_call(
        _tc0_body,
        grid=(N // BLK,),
        in_specs=[
            pl.BlockSpec((BLK, F), lambda i: (i, 0)),
            pl.BlockSpec((BLK, 1), lambda i: (i, 0)),
            pl.BlockSpec((G, 8), lambda i: (0, 0)),
            pl.BlockSpec((F, H), lambda i: (0, 0)),
            pl.BlockSpec((8, H), lambda i: (0, 0)),
            pl.BlockSpec((F, H), lambda i: (0, 0)),
            pl.BlockSpec((8, H), lambda i: (0, 0)),
            pl.BlockSpec((1, H), lambda i: (0, 0)),
        ],
        out_specs=[
            pl.BlockSpec((BLK, H), lambda i: (i, 0)),
            pl.BlockSpec((BLK, HH), lambda i: (i, 0)),
            pl.BlockSpec((BLK, HH), lambda i: (i, 0)),
        ],
        out_shape=[
            jax.ShapeDtypeStruct((N, H), jnp.float32),
            jax.ShapeDtypeStruct((N, HH), jnp.float32),
            jax.ShapeDtypeStruct((N, HH), jnp.float32),
        ],
    )(x, bat2, u_pad, wrx, wru, wox, wou, b2)


def _tcmid_body(hlo_r, hhi_r, wrel_a_r, wrel_b_r, wroot_a_r, wroot_b_r, b_r,
                p_r, rlo_r, rhi_r):
    hlo = hlo_r[...]
    hhi = hhi_r[...]
    p = (jnp.dot(hlo, wrel_a_r[...], preferred_element_type=jnp.float32)
         + jnp.dot(hhi, wrel_b_r[...], preferred_element_type=jnp.float32))
    r = (jnp.dot(hlo, wroot_a_r[...], preferred_element_type=jnp.float32)
         + jnp.dot(hhi, wroot_b_r[...], preferred_element_type=jnp.float32)
         + b_r[...])
    p_r[...] = p
    rlo_r[...] = r[:, :HH]
    rhi_r[...] = r[:, HH:]


def _tcmid(hlo, hhi, wrel_a, wrel_b, wroot_a, wroot_b, b2):
    return pl.pallas_call(
        _tcmid_body,
        grid=(N // BLK,),
        in_specs=[
            pl.BlockSpec((BLK, HH), lambda i: (i, 0)),
            pl.BlockSpec((BLK, HH), lambda i: (i, 0)),
            pl.BlockSpec((HH, H), lambda i: (0, 0)),
            pl.BlockSpec((HH, H), lambda i: (0, 0)),
            pl.BlockSpec((HH, H), lambda i: (0, 0)),
            pl.BlockSpec((HH, H), lambda i: (0, 0)),
            pl.BlockSpec((1, H), lambda i: (0, 0)),
        ],
        out_specs=[
            pl.BlockSpec((BLK, H), lambda i: (i, 0)),
            pl.BlockSpec((BLK, HH), lambda i: (i, 0)),
            pl.BlockSpec((BLK, HH), lambda i: (i, 0)),
        ],
        out_shape=[
            jax.ShapeDtypeStruct((N, H), jnp.float32),
            jax.ShapeDtypeStruct((N, HH), jnp.float32),
            jax.ShapeDtypeStruct((N, HH), jnp.float32),
        ],
    )(hlo, hhi, wrel_a, wrel_b, wroot_a, wroot_b, b2)


def kernel(x, edge_index, u, batch,
           W_rel0, b0, W_root0,
           W_rel1, b1, W_root1,
           W_rel2, b2, W_root2):
    src = edge_index[0]
    dst = edge_index[1]
    gidx = jnp.stack([src * 2, src * 2 + 1]).reshape(2, NS, NCHUNK, CH)
    dsts = dst.reshape(NS, NCHUNK, CH)

    bat2 = batch.reshape(N, 1)
    u_pad = jnp.pad(u, ((0, 0), (0, 5)))
    wrx, wru = W_rel0[:F], jnp.pad(W_rel0[F:], ((0, 5), (0, 0)))
    wox, wou = W_root0[:F], jnp.pad(W_root0[F:], ((0, 5), (0, 0)))

    p, rlo, rhi = _tc0(x, bat2, u_pad, wrx, wru, wox, wou, b0.reshape(1, H))
    hlo, hhi = _sc_segsum(p.reshape(2 * N, HH), gidx, dsts, rlo, rhi)

    p, rlo, rhi = _tcmid(hlo, hhi, W_rel1[:HH], W_rel1[HH:],
                         W_root1[:HH], W_root1[HH:], b1.reshape(1, H))
    hlo, hhi = _sc_segsum(p.reshape(2 * N, HH), gidx, dsts, rlo, rhi)

    p, rlo, rhi = _tcmid(hlo, hhi, W_rel2[:HH], W_rel2[HH:],
                         W_root2[:HH], W_root2[HH:], b2.reshape(1, H))
    olo, ohi = _sc_segsum(p.reshape(2 * N, HH), gidx, dsts, rlo, rhi)

    return jnp.concatenate([olo, ohi], axis=1)


# speedup vs baseline: 6.3843x; 1.9082x over previous
import functools

import jax
import jax.numpy as jnp
from jax import lax
from jax.experimental import pallas as pl
from jax.experimental.pallas import tpu as pltpu
from jax.experimental.pallas import tpu_sc as plsc

N = 10000
E = 320000
F = 128
H = 256
G = 16
HH = H // 2
NS = 16
EPS = E // NS
CH = 80
NCHUNK = EPS // CH
GRP = 50
RPS = 624
REM = N - NS * RPS
REM0 = NS * RPS
BLK = 2000


def _sc_body(p2, gidx, dsts, r_lo, r_hi, out_lo, out_hi,
             agg, idx_v, dst_v, rows_a, rows_b, sem_a, sem_b):
    c = lax.axis_index("c")
    s = lax.axis_index("s")
    row0 = s * RPS

    @pl.when(c == 0)
    def _():
        pltpu.sync_copy(r_lo.at[pl.ds(row0, RPS)], agg.at[pl.ds(row0, RPS)])

        @pl.when(s == NS - 1)
        def _():
            pltpu.sync_copy(r_lo.at[pl.ds(REM0, REM)], agg.at[pl.ds(REM0, REM)])

    @pl.when(c == 1)
    def _():
        pltpu.sync_copy(r_hi.at[pl.ds(row0, RPS)], agg.at[pl.ds(row0, RPS)])

        @pl.when(s == NS - 1)
        def _():
            pltpu.sync_copy(r_hi.at[pl.ds(REM0, REM)], agg.at[pl.ds(REM0, REM)])

    plsc.subcore_barrier()

    for grp in range(NCHUNK // GRP):
        pltpu.sync_copy(gidx.at[c, s, grp], idx_v)
        pltpu.sync_copy(dsts.at[s, grp], dst_v)
        pltpu.async_copy(p2.at[idx_v.at[0]], rows_a, sem_a)

        def chunk2(j, carry):
            k0 = 2 * j
            pltpu.make_async_copy(p2.at[idx_v.at[k0]], rows_a, sem_a).wait()
            pltpu.async_copy(p2.at[idx_v.at[k0 + 1]], rows_b, sem_b)
            pltpu.sync_copy(rows_a, agg.at[dst_v.at[k0]], add=True)
            pltpu.make_async_copy(p2.at[idx_v.at[k0 + 1]], rows_b, sem_b).wait()

            @pl.when(j < GRP // 2 - 1)
            def _():
                pltpu.async_copy(p2.at[idx_v.at[k0 + 2]], rows_a, sem_a)

            pltpu.sync_copy(rows_b, agg.at[dst_v.at[k0 + 1]], add=True)
            return carry

        lax.fori_loop(0, GRP // 2, chunk2, 0)
    plsc.subcore_barrier()

    @pl.when(c == 0)
    def _():
        pltpu.sync_copy(agg.at[pl.ds(row0, RPS)], out_lo.at[pl.ds(row0, RPS)])

        @pl.when(s == NS - 1)
        def _():
            pltpu.sync_copy(agg.at[pl.ds(REM0, REM)], out_lo.at[pl.ds(REM0, REM)])

    @pl.when(c == 1)
    def _():
        pltpu.sync_copy(agg.at[pl.ds(row0, RPS)], out_hi.at[pl.ds(row0, RPS)])

        @pl.when(s == NS - 1)
        def _():
            pltpu.sync_copy(agg.at[pl.ds(REM0, REM)], out_hi.at[pl.ds(REM0, REM)])


_sc_segsum = functools.partial(
    pl.kernel,
    out_type=(
        jax.ShapeDtypeStruct((N, HH), jnp.float32),
        jax.ShapeDtypeStruct((N, HH), jnp.float32),
    ),
    mesh=plsc.VectorSubcoreMesh(core_axis_name="c", subcore_axis_name="s"),
    scratch_types=[
        pltpu.VMEM_SHARED((N, HH), jnp.float32),
        pltpu.VMEM((GRP, CH), jnp.int32),
        pltpu.VMEM((GRP, CH), jnp.int32),
        pltpu.VMEM((CH, HH), jnp.float32),
        pltpu.VMEM((CH, HH), jnp.float32),
        pltpu.SemaphoreType.DMA,
        pltpu.SemaphoreType.DMA,
    ],
)(_sc_body)


def _tc0_body(x_r, bat_r, u_r, wrx_r, wru_r, wox_r, wou_r, b_r,
              p_r, rlo_r, rhi_r):
    xb = x_r[...]
    oh = (bat_r[...] == lax.broadcasted_iota(jnp.int32, (1, G), 1)
          ).astype(jnp.float32)
    uw_rel = jnp.dot(u_r[...], wru_r[...], preferred_element_type=jnp.float32)
    uw_root = jnp.dot(u_r[...], wou_r[...], preferred_element_type=jnp.float32)
    p = (jnp.dot(xb, wrx_r[...], preferred_element_type=jnp.float32)
         + jnp.dot(oh, uw_rel, preferred_element_type=jnp.float32))
    r = (jnp.dot(xb, wox_r[...], preferred_element_type=jnp.float32)
         + jnp.dot(oh, uw_root, preferred_element_type=jnp.float32)
         + b_r[...])
    p_r[...] = p
    rlo_r[...] = r[:, :HH]
    rhi_r[...] = r[:, HH:]


def _tc0(x, bat2, u_pad, wrx, wru, wox, wou, b2):
    return pl.pallas_call(
        _tc0_body,
        grid=(N // BLK,),
        in_specs=[
            pl.BlockSpec((BLK, F), lambda i: (i, 0)),
            pl.BlockSpec((BLK, 1), lambda i: (i, 0)),
            pl.BlockSpec((G, 8), lambda i: (0, 0)),
            pl.BlockSpec((F, H), lambda i: (0, 0)),
            pl.BlockSpec((8, H), lambda i: (0, 0)),
            pl.BlockSpec((F, H), lambda i: (0, 0)),
            pl.BlockSpec((8, H), lambda i: (0, 0)),
            pl.BlockSpec((1, H), lambda i: (0, 0)),
        ],
        out_specs=[
            pl.BlockSpec((BLK, H), lambda i: (i, 0)),
            pl.BlockSpec((BLK, HH), lambda i: (i, 0)),
            pl.BlockSpec((BLK, HH), lambda i: (i, 0)),
        ],
        out_shape=[
            jax.ShapeDtypeStruct((N, H), jnp.float32),
            jax.ShapeDtypeStruct((N, HH), jnp.float32),
            jax.ShapeDtypeStruct((N, HH), jnp.float32),
        ],
    )(x, bat2, u_pad, wrx, wru, wox, wou, b2)


def _tcmid_body(hlo_r, hhi_r, wrel_a_r, wrel_b_r, wroot_a_r, wroot_b_r, b_r,
                p_r, rlo_r, rhi_r):
    hlo = hlo_r[...]
    hhi = hhi_r[...]
    p = (jnp.dot(hlo, wrel_a_r[...], preferred_element_type=jnp.float32)
         + jnp.dot(hhi, wrel_b_r[...], preferred_element_type=jnp.float32))
    r = (jnp.dot(hlo, wroot_a_r[...], preferred_element_type=jnp.float32)
         + jnp.dot(hhi, wroot_b_r[...], preferred_element_type=jnp.float32)
         + b_r[...])
    p_r[...] = p
    rlo_r[...] = r[:, :HH]
    rhi_r[...] = r[:, HH:]


def _tcmid(hlo, hhi, wrel_a, wrel_b, wroot_a, wroot_b, b2):
    return pl.pallas_call(
        _tcmid_body,
        grid=(N // BLK,),
        in_specs=[
            pl.BlockSpec((BLK, HH), lambda i: (i, 0)),
            pl.BlockSpec((BLK, HH), lambda i: (i, 0)),
            pl.BlockSpec((HH, H), lambda i: (0, 0)),
            pl.BlockSpec((HH, H), lambda i: (0, 0)),
            pl.BlockSpec((HH, H), lambda i: (0, 0)),
            pl.BlockSpec((HH, H), lambda i: (0, 0)),
            pl.BlockSpec((1, H), lambda i: (0, 0)),
        ],
        out_specs=[
            pl.BlockSpec((BLK, H), lambda i: (i, 0)),
            pl.BlockSpec((BLK, HH), lambda i: (i, 0)),
            pl.BlockSpec((BLK, HH), lambda i: (i, 0)),
        ],
        out_shape=[
            jax.ShapeDtypeStruct((N, H), jnp.float32),
            jax.ShapeDtypeStruct((N, HH), jnp.float32),
            jax.ShapeDtypeStruct((N, HH), jnp.float32),
        ],
    )(hlo, hhi, wrel_a, wrel_b, wroot_a, wroot_b, b2)


def kernel(x, edge_index, u, batch,
           W_rel0, b0, W_root0,
           W_rel1, b1, W_root1,
           W_rel2, b2, W_root2):
    src = edge_index[0]
    dst = edge_index[1]
    gidx = jnp.stack([src * 2, src * 2 + 1]).reshape(
        2, NS, NCHUNK // GRP, GRP, CH)
    dsts = dst.reshape(NS, NCHUNK // GRP, GRP, CH)

    bat2 = batch.reshape(N, 1)
    u_pad = jnp.pad(u, ((0, 0), (0, 5)))
    wrx, wru = W_rel0[:F], jnp.pad(W_rel0[F:], ((0, 5), (0, 0)))
    wox, wou = W_root0[:F], jnp.pad(W_root0[F:], ((0, 5), (0, 0)))

    p, rlo, rhi = _tc0(x, bat2, u_pad, wrx, wru, wox, wou, b0.reshape(1, H))
    hlo, hhi = _sc_segsum(p.reshape(2 * N, HH), gidx, dsts, rlo, rhi)

    p, rlo, rhi = _tcmid(hlo, hhi, W_rel1[:HH], W_rel1[HH:],
                         W_root1[:HH], W_root1[HH:], b1.reshape(1, H))
    hlo, hhi = _sc_segsum(p.reshape(2 * N, HH), gidx, dsts, rlo, rhi)

    p, rlo, rhi = _tcmid(hlo, hhi, W_rel2[:HH], W_rel2[HH:],
                         W_root2[:HH], W_root2[HH:], b2.reshape(1, H))
    olo, ohi = _sc_segsum(p.reshape(2 * N, HH), gidx, dsts, rlo, rhi)

    return jnp.concatenate([olo, ohi], axis=1)


# speedup vs baseline: 7.5703x; 1.1858x over previous
import functools

import jax
import jax.numpy as jnp
from jax import lax
from jax.experimental import pallas as pl
from jax.experimental.pallas import tpu as pltpu
from jax.experimental.pallas import tpu_sc as plsc

N = 10000
E = 320000
F = 128
H = 256
G = 16
HH = H // 2
NS = 16
EPS = E // NS
CH = 128
EPT = 20480
NCHUNK = EPT // CH
GRP = 32
NPAD = 8
RPS = 624
REM = N - NS * RPS
REM0 = NS * RPS
BLK = 2000


def _sc_body(p2, gidx, dsts, r_lo, r_hi, out_lo, out_hi,
             agg, idx_v, dst_v, rows_a, rows_b, sem_a, sem_b):
    c = lax.axis_index("c")
    s = lax.axis_index("s")
    row0 = s * RPS

    @pl.when(c == 0)
    def _():
        pltpu.sync_copy(r_lo.at[pl.ds(row0, RPS)], agg.at[pl.ds(row0, RPS)])

        @pl.when(s == NS - 1)
        def _():
            pltpu.sync_copy(r_lo.at[pl.ds(REM0, REM)], agg.at[pl.ds(REM0, REM)])

    @pl.when(c == 1)
    def _():
        pltpu.sync_copy(r_hi.at[pl.ds(row0, RPS)], agg.at[pl.ds(row0, RPS)])

        @pl.when(s == NS - 1)
        def _():
            pltpu.sync_copy(r_hi.at[pl.ds(REM0, REM)], agg.at[pl.ds(REM0, REM)])

    plsc.subcore_barrier()

    for grp in range(NCHUNK // GRP):
        pltpu.sync_copy(gidx.at[c, s, grp], idx_v)
        pltpu.sync_copy(dsts.at[s, grp], dst_v)
        pltpu.async_copy(p2.at[idx_v.at[0]], rows_a, sem_a)

        def chunk2(j, carry):
            k0 = 2 * j
            pltpu.make_async_copy(p2.at[idx_v.at[k0]], rows_a, sem_a).wait()
            pltpu.async_copy(p2.at[idx_v.at[k0 + 1]], rows_b, sem_b)
            pltpu.sync_copy(rows_a, agg.at[dst_v.at[k0]], add=True)
            pltpu.make_async_copy(p2.at[idx_v.at[k0 + 1]], rows_b, sem_b).wait()

            @pl.when(j < GRP // 2 - 1)
            def _():
                pltpu.async_copy(p2.at[idx_v.at[k0 + 2]], rows_a, sem_a)

            pltpu.sync_copy(rows_b, agg.at[dst_v.at[k0 + 1]], add=True)
            return carry

        lax.fori_loop(0, GRP // 2, chunk2, 0)
    plsc.subcore_barrier()

    @pl.when(c == 0)
    def _():
        pltpu.sync_copy(agg.at[pl.ds(row0, RPS)], out_lo.at[pl.ds(row0, RPS)])

        @pl.when(s == NS - 1)
        def _():
            pltpu.sync_copy(agg.at[pl.ds(REM0, REM)], out_lo.at[pl.ds(REM0, REM)])

    @pl.when(c == 1)
    def _():
        pltpu.sync_copy(agg.at[pl.ds(row0, RPS)], out_hi.at[pl.ds(row0, RPS)])

        @pl.when(s == NS - 1)
        def _():
            pltpu.sync_copy(agg.at[pl.ds(REM0, REM)], out_hi.at[pl.ds(REM0, REM)])


_sc_segsum = functools.partial(
    pl.kernel,
    out_type=(
        jax.ShapeDtypeStruct((N, HH), jnp.float32),
        jax.ShapeDtypeStruct((N, HH), jnp.float32),
    ),
    mesh=plsc.VectorSubcoreMesh(core_axis_name="c", subcore_axis_name="s"),
    scratch_types=[
        pltpu.VMEM_SHARED((N + NPAD, HH), jnp.float32),
        pltpu.VMEM((GRP, CH), jnp.int32),
        pltpu.VMEM((GRP, CH), jnp.int32),
        pltpu.VMEM((CH, HH), jnp.float32),
        pltpu.VMEM((CH, HH), jnp.float32),
        pltpu.SemaphoreType.DMA,
        pltpu.SemaphoreType.DMA,
    ],
)(_sc_body)


def _tc0_body(x_r, bat_r, u_r, wrx_r, wru_r, wox_r, wou_r, b_r,
              p_r, rlo_r, rhi_r):
    xb = x_r[...]
    oh = (bat_r[...] == lax.broadcasted_iota(jnp.int32, (1, G), 1)
          ).astype(jnp.float32)
    uw_rel = jnp.dot(u_r[...], wru_r[...], preferred_element_type=jnp.float32)
    uw_root = jnp.dot(u_r[...], wou_r[...], preferred_element_type=jnp.float32)
    p = (jnp.dot(xb, wrx_r[...], preferred_element_type=jnp.float32)
         + jnp.dot(oh, uw_rel, preferred_element_type=jnp.float32))
    r = (jnp.dot(xb, wox_r[...], preferred_element_type=jnp.float32)
         + jnp.dot(oh, uw_root, preferred_element_type=jnp.float32)
         + b_r[...])
    p_r[...] = p
    rlo_r[...] = r[:, :HH]
    rhi_r[...] = r[:, HH:]


def _tc0(x, bat2, u_pad, wrx, wru, wox, wou, b2):
    return pl.pallas_call(
        _tc0_body,
        grid=(N // BLK,),
        in_specs=[
            pl.BlockSpec((BLK, F), lambda i: (i, 0)),
            pl.BlockSpec((BLK, 1), lambda i: (i, 0)),
            pl.BlockSpec((G, 8), lambda i: (0, 0)),
            pl.BlockSpec((F, H), lambda i: (0, 0)),
            pl.BlockSpec((8, H), lambda i: (0, 0)),
            pl.BlockSpec((F, H), lambda i: (0, 0)),
            pl.BlockSpec((8, H), lambda i: (0, 0)),
            pl.BlockSpec((1, H), lambda i: (0, 0)),
        ],
        out_specs=[
            pl.BlockSpec((BLK, H), lambda i: (i, 0)),
            pl.BlockSpec((BLK, HH), lambda i: (i, 0)),
            pl.BlockSpec((BLK, HH), lambda i: (i, 0)),
        ],
        out_shape=[
            jax.ShapeDtypeStruct((N, H), jnp.float32),
            jax.ShapeDtypeStruct((N, HH), jnp.float32),
            jax.ShapeDtypeStruct((N, HH), jnp.float32),
        ],
    )(x, bat2, u_pad, wrx, wru, wox, wou, b2)


def _tcmid_body(hlo_r, hhi_r, wrel_a_r, wrel_b_r, wroot_a_r, wroot_b_r, b_r,
                p_r, rlo_r, rhi_r):
    hlo = hlo_r[...]
    hhi = hhi_r[...]
    p = (jnp.dot(hlo, wrel_a_r[...], preferred_element_type=jnp.float32)
         + jnp.dot(hhi, wrel_b_r[...], preferred_element_type=jnp.float32))
    r = (jnp.dot(hlo, wroot_a_r[...], preferred_element_type=jnp.float32)
         + jnp.dot(hhi, wroot_b_r[...], preferred_element_type=jnp.float32)
         + b_r[...])
    p_r[...] = p
    rlo_r[...] = r[:, :HH]
    rhi_r[...] = r[:, HH:]


def _tcmid(hlo, hhi, wrel_a, wrel_b, wroot_a, wroot_b, b2):
    return pl.pallas_call(
        _tcmid_body,
        grid=(N // BLK,),
        in_specs=[
            pl.BlockSpec((BLK, HH), lambda i: (i, 0)),
            pl.BlockSpec((BLK, HH), lambda i: (i, 0)),
            pl.BlockSpec((HH, H), lambda i: (0, 0)),
            pl.BlockSpec((HH, H), lambda i: (0, 0)),
            pl.BlockSpec((HH, H), lambda i: (0, 0)),
            pl.BlockSpec((HH, H), lambda i: (0, 0)),
            pl.BlockSpec((1, H), lambda i: (0, 0)),
        ],
        out_specs=[
            pl.BlockSpec((BLK, H), lambda i: (i, 0)),
            pl.BlockSpec((BLK, HH), lambda i: (i, 0)),
            pl.BlockSpec((BLK, HH), lambda i: (i, 0)),
        ],
        out_shape=[
            jax.ShapeDtypeStruct((N, H), jnp.float32),
            jax.ShapeDtypeStruct((N, HH), jnp.float32),
            jax.ShapeDtypeStruct((N, HH), jnp.float32),
        ],
    )(hlo, hhi, wrel_a, wrel_b, wroot_a, wroot_b, b2)


def kernel(x, edge_index, u, batch,
           W_rel0, b0, W_root0,
           W_rel1, b1, W_root1,
           W_rel2, b2, W_root2):
    pad = EPT - EPS
    src_p = jnp.pad(edge_index[0].reshape(NS, EPS), ((0, 0), (0, pad)))
    src_p = src_p.at[:, EPS:].set(jnp.arange(pad, dtype=jnp.int32)[None, :] * 41 % N)
    dst_p = jnp.pad(edge_index[1].reshape(NS, EPS), ((0, 0), (0, pad)))
    dst_p = dst_p.at[:, EPS:].set(N + jnp.arange(pad, dtype=jnp.int32)[None, :] % NPAD)
    gidx = jnp.stack([src_p * 2, src_p * 2 + 1]).reshape(
        2, NS, NCHUNK // GRP, GRP, CH)
    dsts = dst_p.reshape(NS, NCHUNK // GRP, GRP, CH)

    bat2 = batch.reshape(N, 1)
    u_pad = jnp.pad(u, ((0, 0), (0, 5)))
    wrx, wru = W_rel0[:F], jnp.pad(W_rel0[F:], ((0, 5), (0, 0)))
    wox, wou = W_root0[:F], jnp.pad(W_root0[F:], ((0, 5), (0, 0)))

    p, rlo, rhi = _tc0(x, bat2, u_pad, wrx, wru, wox, wou, b0.reshape(1, H))
    hlo, hhi = _sc_segsum(p.reshape(2 * N, HH), gidx, dsts, rlo, rhi)

    p, rlo, rhi = _tcmid(hlo, hhi, W_rel1[:HH], W_rel1[HH:],
                         W_root1[:HH], W_root1[HH:], b1.reshape(1, H))
    hlo, hhi = _sc_segsum(p.reshape(2 * N, HH), gidx, dsts, rlo, rhi)

    p, rlo, rhi = _tcmid(hlo, hhi, W_rel2[:HH], W_rel2[HH:],
                         W_root2[:HH], W_root2[HH:], b2.reshape(1, H))
    olo, ohi = _sc_segsum(p.reshape(2 * N, HH), gidx, dsts, rlo, rhi)

    return jnp.concatenate([olo, ohi], axis=1)


# speedup vs baseline: 7.9932x; 1.0559x over previous
import functools

import jax
import jax.numpy as jnp
from jax import lax
from jax.experimental import pallas as pl
from jax.experimental.pallas import tpu as pltpu
from jax.experimental.pallas import tpu_sc as plsc

N = 10000
E = 320000
F = 128
H = 256
G = 16
HH = H // 2
NS = 16
EPS = E // NS
CH = 64
EPT = 20480
NCHUNK = EPT // CH
GRP = 40
NBUF = 4
NPAD = 8
RPS = 624
REM = N - NS * RPS
REM0 = NS * RPS
BLK = 2000


def _sc_body(p2, gidx, dsts, r_lo, r_hi, out_lo, out_hi,
             agg, idx_v, dst_v, rows, gsems, ssems):
    c = lax.axis_index("c")
    s = lax.axis_index("s")
    row0 = s * RPS

    @pl.when(c == 0)
    def _():
        pltpu.sync_copy(r_lo.at[pl.ds(row0, RPS)], agg.at[pl.ds(row0, RPS)])

        @pl.when(s == NS - 1)
        def _():
            pltpu.sync_copy(r_lo.at[pl.ds(REM0, REM)], agg.at[pl.ds(REM0, REM)])

    @pl.when(c == 1)
    def _():
        pltpu.sync_copy(r_hi.at[pl.ds(row0, RPS)], agg.at[pl.ds(row0, RPS)])

        @pl.when(s == NS - 1)
        def _():
            pltpu.sync_copy(r_hi.at[pl.ds(REM0, REM)], agg.at[pl.ds(REM0, REM)])

    plsc.subcore_barrier()

    for grp in range(NCHUNK // GRP):
        pltpu.sync_copy(gidx.at[c, s, grp], idx_v)
        pltpu.sync_copy(dsts.at[s, grp], dst_v)
        for o in range(NBUF):
            pltpu.async_copy(p2.at[idx_v.at[o]], rows.at[o], gsems.at[o])

        def chunkn(j, carry):
            k0 = NBUF * j
            for o in range(NBUF):
                pltpu.make_async_copy(
                    p2.at[idx_v.at[k0 + o]], rows.at[o], gsems.at[o]).wait()
                pltpu.async_copy(
                    rows.at[o], agg.at[dst_v.at[k0 + o]], ssems.at[o], add=True)
            for o in range(NBUF):
                pltpu.make_async_copy(
                    rows.at[o], agg.at[dst_v.at[k0 + o]], ssems.at[o]).wait()

                @pl.when(j < GRP // NBUF - 1)
                def _(o=o, k0=k0):
                    pltpu.async_copy(
                        p2.at[idx_v.at[k0 + o + NBUF]], rows.at[o], gsems.at[o])

            return carry

        lax.fori_loop(0, GRP // NBUF, chunkn, 0)
    plsc.subcore_barrier()

    @pl.when(c == 0)
    def _():
        pltpu.sync_copy(agg.at[pl.ds(row0, RPS)], out_lo.at[pl.ds(row0, RPS)])

        @pl.when(s == NS - 1)
        def _():
            pltpu.sync_copy(agg.at[pl.ds(REM0, REM)], out_lo.at[pl.ds(REM0, REM)])

    @pl.when(c == 1)
    def _():
        pltpu.sync_copy(agg.at[pl.ds(row0, RPS)], out_hi.at[pl.ds(row0, RPS)])

        @pl.when(s == NS - 1)
        def _():
            pltpu.sync_copy(agg.at[pl.ds(REM0, REM)], out_hi.at[pl.ds(REM0, REM)])


_sc_segsum = functools.partial(
    pl.kernel,
    out_type=(
        jax.ShapeDtypeStruct((N, HH), jnp.float32),
        jax.ShapeDtypeStruct((N, HH), jnp.float32),
    ),
    mesh=plsc.VectorSubcoreMesh(core_axis_name="c", subcore_axis_name="s"),
    scratch_types=[
        pltpu.VMEM_SHARED((N + NPAD, HH), jnp.float32),
        pltpu.VMEM((GRP, CH), jnp.int32),
        pltpu.VMEM((GRP, CH), jnp.int32),
        pltpu.VMEM((NBUF, CH, HH), jnp.float32),
        pltpu.SemaphoreType.DMA((NBUF,)),
        pltpu.SemaphoreType.DMA((NBUF,)),
    ],
)(_sc_body)


def _tc0_body(x_r, bat_r, u_r, wrx_r, wru_r, wox_r, wou_r, b_r,
              p_r, rlo_r, rhi_r):
    xb = x_r[...]
    oh = (bat_r[...] == lax.broadcasted_iota(jnp.int32, (1, G), 1)
          ).astype(jnp.float32)
    uw_rel = jnp.dot(u_r[...], wru_r[...], preferred_element_type=jnp.float32)
    uw_root = jnp.dot(u_r[...], wou_r[...], preferred_element_type=jnp.float32)
    p = (jnp.dot(xb, wrx_r[...], preferred_element_type=jnp.float32)
         + jnp.dot(oh, uw_rel, preferred_element_type=jnp.float32))
    r = (jnp.dot(xb, wox_r[...], preferred_element_type=jnp.float32)
         + jnp.dot(oh, uw_root, preferred_element_type=jnp.float32)
         + b_r[...])
    p_r[...] = p
    rlo_r[...] = r[:, :HH]
    rhi_r[...] = r[:, HH:]


def _tc0(x, bat2, u_pad, wrx, wru, wox, wou, b2):
    return pl.pallas_call(
        _tc0_body,
        grid=(N // BLK,),
        in_specs=[
            pl.BlockSpec((BLK, F), lambda i: (i, 0)),
            pl.BlockSpec((BLK, 1), lambda i: (i, 0)),
            pl.BlockSpec((G, 8), lambda i: (0, 0)),
            pl.BlockSpec((F, H), lambda i: (0, 0)),
            pl.BlockSpec((8, H), lambda i: (0, 0)),
            pl.BlockSpec((F, H), lambda i: (0, 0)),
            pl.BlockSpec((8, H), lambda i: (0, 0)),
            pl.BlockSpec((1, H), lambda i: (0, 0)),
        ],
        out_specs=[
            pl.BlockSpec((BLK, H), lambda i: (i, 0)),
            pl.BlockSpec((BLK, HH), lambda i: (i, 0)),
            pl.BlockSpec((BLK, HH), lambda i: (i, 0)),
        ],
        out_shape=[
            jax.ShapeDtypeStruct((N, H), jnp.float32),
            jax.ShapeDtypeStruct((N, HH), jnp.float32),
            jax.ShapeDtypeStruct((N, HH), jnp.float32),
        ],
    )(x, bat2, u_pad, wrx, wru, wox, wou, b2)


def _tcmid_body(hlo_r, hhi_r, wrel_a_r, wrel_b_r, wroot_a_r, wroot_b_r, b_r,
                p_r, rlo_r, rhi_r):
    hlo = hlo_r[...]
    hhi = hhi_r[...]
    p = (jnp.dot(hlo, wrel_a_r[...], preferred_element_type=jnp.float32)
         + jnp.dot(hhi, wrel_b_r[...], preferred_element_type=jnp.float32))
    r = (jnp.dot(hlo, wroot_a_r[...], preferred_element_type=jnp.float32)
         + jnp.dot(hhi, wroot_b_r[...], preferred_element_type=jnp.float32)
         + b_r[...])
    p_r[...] = p
    rlo_r[...] = r[:, :HH]
    rhi_r[...] = r[:, HH:]


def _tcmid(hlo, hhi, wrel_a, wrel_b, wroot_a, wroot_b, b2):
    return pl.pallas_call(
        _tcmid_body,
        grid=(N // BLK,),
        in_specs=[
            pl.BlockSpec((BLK, HH), lambda i: (i, 0)),
            pl.BlockSpec((BLK, HH), lambda i: (i, 0)),
            pl.BlockSpec((HH, H), lambda i: (0, 0)),
            pl.BlockSpec((HH, H), lambda i: (0, 0)),
            pl.BlockSpec((HH, H), lambda i: (0, 0)),
            pl.BlockSpec((HH, H), lambda i: (0, 0)),
            pl.BlockSpec((1, H), lambda i: (0, 0)),
        ],
        out_specs=[
            pl.BlockSpec((BLK, H), lambda i: (i, 0)),
            pl.BlockSpec((BLK, HH), lambda i: (i, 0)),
            pl.BlockSpec((BLK, HH), lambda i: (i, 0)),
        ],
        out_shape=[
            jax.ShapeDtypeStruct((N, H), jnp.float32),
            jax.ShapeDtypeStruct((N, HH), jnp.float32),
            jax.ShapeDtypeStruct((N, HH), jnp.float32),
        ],
    )(hlo, hhi, wrel_a, wrel_b, wroot_a, wroot_b, b2)


def kernel(x, edge_index, u, batch,
           W_rel0, b0, W_root0,
           W_rel1, b1, W_root1,
           W_rel2, b2, W_root2):
    pad = EPT - EPS
    src_p = jnp.pad(edge_index[0].reshape(NS, EPS), ((0, 0), (0, pad)))
    src_p = src_p.at[:, EPS:].set(jnp.arange(pad, dtype=jnp.int32)[None, :] * 41 % N)
    dst_p = jnp.pad(edge_index[1].reshape(NS, EPS), ((0, 0), (0, pad)))
    dst_p = dst_p.at[:, EPS:].set(N + jnp.arange(pad, dtype=jnp.int32)[None, :] % NPAD)
    gidx = jnp.stack([src_p * 2, src_p * 2 + 1]).reshape(
        2, NS, NCHUNK // GRP, GRP, CH)
    dsts = dst_p.reshape(NS, NCHUNK // GRP, GRP, CH)

    bat2 = batch.reshape(N, 1)
    u_pad = jnp.pad(u, ((0, 0), (0, 5)))
    wrx, wru = W_rel0[:F], jnp.pad(W_rel0[F:], ((0, 5), (0, 0)))
    wox, wou = W_root0[:F], jnp.pad(W_root0[F:], ((0, 5), (0, 0)))

    p, rlo, rhi = _tc0(x, bat2, u_pad, wrx, wru, wox, wou, b0.reshape(1, H))
    hlo, hhi = _sc_segsum(p.reshape(2 * N, HH), gidx, dsts, rlo, rhi)

    p, rlo, rhi = _tcmid(hlo, hhi, W_rel1[:HH], W_rel1[HH:],
                         W_root1[:HH], W_root1[HH:], b1.reshape(1, H))
    hlo, hhi = _sc_segsum(p.reshape(2 * N, HH), gidx, dsts, rlo, rhi)

    p, rlo, rhi = _tcmid(hlo, hhi, W_rel2[:HH], W_rel2[HH:],
                         W_root2[:HH], W_root2[HH:], b2.reshape(1, H))
    olo, ohi = _sc_segsum(p.reshape(2 * N, HH), gidx, dsts, rlo, rhi)

    return jnp.concatenate([olo, ohi], axis=1)


# speedup vs baseline: 8.2588x; 1.0332x over previous
import functools

import jax
import jax.numpy as jnp
from jax import lax
from jax.experimental import pallas as pl
from jax.experimental.pallas import tpu as pltpu
from jax.experimental.pallas import tpu_sc as plsc

N = 10000
E = 320000
F = 128
H = 256
G = 16
HH = H // 2
NS = 16
EPS = E // NS
CH = 64
EPT = 20480
NCHUNK = EPT // CH
GRP = 64
NBUF = 4
NPAD = 8
RPS = 624
REM = N - NS * RPS
REM0 = NS * RPS
BLK = 2000


def _sc_body(p2, gidx, dsts, r_lo, r_hi, out_lo, out_hi,
             agg, idx_v, dst_v, rows, gsems, ssems):
    c = lax.axis_index("c")
    s = lax.axis_index("s")
    row0 = s * RPS

    pltpu.sync_copy(gidx.at[c, s, 0], idx_v)
    pltpu.sync_copy(dsts.at[s, 0], dst_v)
    for o in range(NBUF):
        pltpu.async_copy(p2.at[idx_v.at[o]], rows.at[o], gsems.at[o])

    @pl.when(c == 0)
    def _():
        pltpu.sync_copy(r_lo.at[pl.ds(row0, RPS)], agg.at[pl.ds(row0, RPS)])

        @pl.when(s == NS - 1)
        def _():
            pltpu.sync_copy(r_lo.at[pl.ds(REM0, REM)], agg.at[pl.ds(REM0, REM)])

    @pl.when(c == 1)
    def _():
        pltpu.sync_copy(r_hi.at[pl.ds(row0, RPS)], agg.at[pl.ds(row0, RPS)])

        @pl.when(s == NS - 1)
        def _():
            pltpu.sync_copy(r_hi.at[pl.ds(REM0, REM)], agg.at[pl.ds(REM0, REM)])

    plsc.subcore_barrier()

    for grp in range(NCHUNK // GRP):
        if grp:
            pltpu.sync_copy(gidx.at[c, s, grp], idx_v)
            pltpu.sync_copy(dsts.at[s, grp], dst_v)
            for o in range(NBUF):
                pltpu.async_copy(p2.at[idx_v.at[o]], rows.at[o], gsems.at[o])

        def chunkn(j, carry):
            k0 = NBUF * j
            for o in range(NBUF):
                pltpu.make_async_copy(
                    p2.at[idx_v.at[k0 + o]], rows.at[o], gsems.at[o]).wait()
                pltpu.async_copy(
                    rows.at[o], agg.at[dst_v.at[k0 + o]], ssems.at[o], add=True)
            for o in range(NBUF):
                pltpu.make_async_copy(
                    rows.at[o], agg.at[dst_v.at[k0 + o]], ssems.at[o]).wait()

                @pl.when(j < GRP // NBUF - 1)
                def _(o=o, k0=k0):
                    pltpu.async_copy(
                        p2.at[idx_v.at[k0 + o + NBUF]], rows.at[o], gsems.at[o])

            return carry

        lax.fori_loop(0, GRP // NBUF, chunkn, 0)
    plsc.subcore_barrier()

    @pl.when(c == 0)
    def _():
        pltpu.sync_copy(agg.at[pl.ds(row0, RPS)], out_lo.at[pl.ds(row0, RPS)])

        @pl.when(s == NS - 1)
        def _():
            pltpu.sync_copy(agg.at[pl.ds(REM0, REM)], out_lo.at[pl.ds(REM0, REM)])

    @pl.when(c == 1)
    def _():
        pltpu.sync_copy(agg.at[pl.ds(row0, RPS)], out_hi.at[pl.ds(row0, RPS)])

        @pl.when(s == NS - 1)
        def _():
            pltpu.sync_copy(agg.at[pl.ds(REM0, REM)], out_hi.at[pl.ds(REM0, REM)])


_sc_segsum = functools.partial(
    pl.kernel,
    out_type=(
        jax.ShapeDtypeStruct((N, HH), jnp.float32),
        jax.ShapeDtypeStruct((N, HH), jnp.float32),
    ),
    mesh=plsc.VectorSubcoreMesh(core_axis_name="c", subcore_axis_name="s"),
    scratch_types=[
        pltpu.VMEM_SHARED((N + NPAD, HH), jnp.float32),
        pltpu.VMEM((GRP, CH), jnp.int32),
        pltpu.VMEM((GRP, CH), jnp.int32),
        pltpu.VMEM((NBUF, CH, HH), jnp.float32),
        pltpu.SemaphoreType.DMA((NBUF,)),
        pltpu.SemaphoreType.DMA((NBUF,)),
    ],
)(_sc_body)


def _tc0_body(x_r, bat_r, u_r, wrx_r, wru_r, wox_r, wou_r, b_r,
              p_r, rlo_r, rhi_r):
    xb = x_r[...]
    oh = (bat_r[...] == lax.broadcasted_iota(jnp.int32, (1, G), 1)
          ).astype(jnp.float32)
    uw_rel = jnp.dot(u_r[...], wru_r[...], preferred_element_type=jnp.float32)
    uw_root = jnp.dot(u_r[...], wou_r[...], preferred_element_type=jnp.float32)
    p = (jnp.dot(xb, wrx_r[...], preferred_element_type=jnp.float32)
         + jnp.dot(oh, uw_rel, preferred_element_type=jnp.float32))
    r = (jnp.dot(xb, wox_r[...], preferred_element_type=jnp.float32)
         + jnp.dot(oh, uw_root, preferred_element_type=jnp.float32)
         + b_r[...])
    p_r[...] = p
    rlo_r[...] = r[:, :HH]
    rhi_r[...] = r[:, HH:]


def _tc0(x, bat2, u_pad, wrx, wru, wox, wou, b2):
    return pl.pallas_call(
        _tc0_body,
        grid=(N // BLK,),
        in_specs=[
            pl.BlockSpec((BLK, F), lambda i: (i, 0)),
            pl.BlockSpec((BLK, 1), lambda i: (i, 0)),
            pl.BlockSpec((G, 8), lambda i: (0, 0)),
            pl.BlockSpec((F, H), lambda i: (0, 0)),
            pl.BlockSpec((8, H), lambda i: (0, 0)),
            pl.BlockSpec((F, H), lambda i: (0, 0)),
            pl.BlockSpec((8, H), lambda i: (0, 0)),
            pl.BlockSpec((1, H), lambda i: (0, 0)),
        ],
        out_specs=[
            pl.BlockSpec((BLK, H), lambda i: (i, 0)),
            pl.BlockSpec((BLK, HH), lambda i: (i, 0)),
            pl.BlockSpec((BLK, HH), lambda i: (i, 0)),
        ],
        out_shape=[
            jax.ShapeDtypeStruct((N, H), jnp.float32),
            jax.ShapeDtypeStruct((N, HH), jnp.float32),
            jax.ShapeDtypeStruct((N, HH), jnp.float32),
        ],
    )(x, bat2, u_pad, wrx, wru, wox, wou, b2)


def _tcmid_body(hlo_r, hhi_r, wrel_a_r, wrel_b_r, wroot_a_r, wroot_b_r, b_r,
                p_r, rlo_r, rhi_r):
    hlo = hlo_r[...]
    hhi = hhi_r[...]
    p = (jnp.dot(hlo, wrel_a_r[...], preferred_element_type=jnp.float32)
         + jnp.dot(hhi, wrel_b_r[...], preferred_element_type=jnp.float32))
    r = (jnp.dot(hlo, wroot_a_r[...], preferred_element_type=jnp.float32)
         + jnp.dot(hhi, wroot_b_r[...], preferred_element_type=jnp.float32)
         + b_r[...])
    p_r[...] = p
    rlo_r[...] = r[:, :HH]
    rhi_r[...] = r[:, HH:]


def _tcmid(hlo, hhi, wrel_a, wrel_b, wroot_a, wroot_b, b2):
    return pl.pallas_call(
        _tcmid_body,
        grid=(N // BLK,),
        in_specs=[
            pl.BlockSpec((BLK, HH), lambda i: (i, 0)),
            pl.BlockSpec((BLK, HH), lambda i: (i, 0)),
            pl.BlockSpec((HH, H), lambda i: (0, 0)),
            pl.BlockSpec((HH, H), lambda i: (0, 0)),
            pl.BlockSpec((HH, H), lambda i: (0, 0)),
            pl.BlockSpec((HH, H), lambda i: (0, 0)),
            pl.BlockSpec((1, H), lambda i: (0, 0)),
        ],
        out_specs=[
            pl.BlockSpec((BLK, H), lambda i: (i, 0)),
            pl.BlockSpec((BLK, HH), lambda i: (i, 0)),
            pl.BlockSpec((BLK, HH), lambda i: (i, 0)),
        ],
        out_shape=[
            jax.ShapeDtypeStruct((N, H), jnp.float32),
            jax.ShapeDtypeStruct((N, HH), jnp.float32),
            jax.ShapeDtypeStruct((N, HH), jnp.float32),
        ],
    )(hlo, hhi, wrel_a, wrel_b, wroot_a, wroot_b, b2)


def kernel(x, edge_index, u, batch,
           W_rel0, b0, W_root0,
           W_rel1, b1, W_root1,
           W_rel2, b2, W_root2):
    pad = EPT - EPS
    src_p = jnp.pad(edge_index[0].reshape(NS, EPS), ((0, 0), (0, pad)))
    src_p = src_p.at[:, EPS:].set(jnp.arange(pad, dtype=jnp.int32)[None, :] * 41 % N)
    dst_p = jnp.pad(edge_index[1].reshape(NS, EPS), ((0, 0), (0, pad)))
    dst_p = dst_p.at[:, EPS:].set(N + jnp.arange(pad, dtype=jnp.int32)[None, :] % NPAD)
    gidx = jnp.stack([src_p * 2, src_p * 2 + 1]).reshape(
        2, NS, NCHUNK // GRP, GRP, CH)
    dsts = dst_p.reshape(NS, NCHUNK // GRP, GRP, CH)

    bat2 = batch.reshape(N, 1)
    u_pad = jnp.pad(u, ((0, 0), (0, 5)))
    wrx, wru = W_rel0[:F], jnp.pad(W_rel0[F:], ((0, 5), (0, 0)))
    wox, wou = W_root0[:F], jnp.pad(W_root0[F:], ((0, 5), (0, 0)))

    p, rlo, rhi = _tc0(x, bat2, u_pad, wrx, wru, wox, wou, b0.reshape(1, H))
    hlo, hhi = _sc_segsum(p.reshape(2 * N, HH), gidx, dsts, rlo, rhi)

    p, rlo, rhi = _tcmid(hlo, hhi, W_rel1[:HH], W_rel1[HH:],
                         W_root1[:HH], W_root1[HH:], b1.reshape(1, H))
    hlo, hhi = _sc_segsum(p.reshape(2 * N, HH), gidx, dsts, rlo, rhi)

    p, rlo, rhi = _tcmid(hlo, hhi, W_rel2[:HH], W_rel2[HH:],
                         W_root2[:HH], W_root2[HH:], b2.reshape(1, H))
    olo, ohi = _sc_segsum(p.reshape(2 * N, HH), gidx, dsts, rlo, rhi)

    return jnp.concatenate([olo, ohi], axis=1)
